# baseline (speedup 1.0000x reference)
import functools

import jax
import jax.numpy as jnp
import numpy as np
from jax import lax
from jax.experimental import pallas as pl
from jax.experimental.pallas import tpu as pltpu
from jax.experimental.pallas import tpu_sc as plsc

NFD = 4
MASK_ID = 1
SOS_ID = 2
EOS_ID = 3
EMB_DIM = 64
HALF = EMB_DIM // 2

_LANES = 16


def _sc_scan_body(tok_hbm, val_hbm, out_hbm, tok_v, val_v, out_v):
    info = plsc.get_sparse_core_info()
    nc = info.num_cores
    wid = lax.axis_index("s") * nc + lax.axis_index("c")
    num_workers = nc * info.num_subcores
    groups = tok_hbm.shape[0]
    g_per_w = groups // num_workers
    seq = tok_hbm.shape[2]

    zf = jnp.zeros((_LANES,), jnp.float32)
    zi = jnp.zeros((_LANES,), jnp.int32)
    one_i = jnp.full((_LANES,), 1, jnp.int32)
    mask_i = jnp.full((_LANES,), MASK_ID, jnp.int32)
    nfd_i = jnp.full((_LANES,), NFD, jnp.int32)
    sos_i = jnp.full((_LANES,), SOS_ID, jnp.int32)
    eos_i = jnp.full((_LANES,), EOS_ID, jnp.int32)

    for gi in range(g_per_w):
        g = wid * g_per_w + gi
        pltpu.sync_copy(tok_hbm.at[g], tok_v)
        pltpu.sync_copy(val_hbm.at[g], val_v)
        for i in range(2):

            def step(s, carry, i=i):
                unk, ck, ci = carry
                t = tok_v[i, s, :]
                vraw = val_v[i, s, :]
                special = t <= nfd_i
                tcv = jnp.where(special, t, nfd_i)
                v = jnp.where(special, zf, vraw)
                sm = special & (t != sos_i) & (t != eos_i)
                unk2 = unk | jnp.where(sm, one_i, zi)
                unkb = unk2 > zi
                ck2 = ck + v
                pk = jnp.where(unkb, zf, ck2)
                ci2 = jnp.where(sm, zf, ci + v)
                tk = jnp.where(unkb & (tcv == nfd_i), mask_i, tcv)
                out_v[i, s, 0, :] = pk
                out_v[i, s, 1, :] = ci2
                out_v[i, s, 2, :] = tk.astype(jnp.float32)
                out_v[i, s, 3, :] = tcv.astype(jnp.float32)
                return (unk2, ck2, ci2)

            lax.fori_loop(0, seq, step, (zi, zf, zf))
        pltpu.sync_copy(out_v, out_hbm.at[g])


def _sc_scan(tok_r, val_r):
    groups, _, seq, lanes = tok_r.shape
    mesh = plsc.VectorSubcoreMesh(core_axis_name="c", subcore_axis_name="s")
    fn = functools.partial(
        pl.kernel,
        mesh=mesh,
        out_type=jax.ShapeDtypeStruct((groups, 2, seq, 4, lanes), jnp.float32),
        scratch_types=[
            pltpu.VMEM((2, seq, lanes), jnp.int32),
            pltpu.VMEM((2, seq, lanes), jnp.float32),
            pltpu.VMEM((2, seq, 4, lanes), jnp.float32),
        ],
        compiler_params=pltpu.CompilerParams(use_tc_tiling_on_sc=False),
    )(_sc_scan_body)
    return fn(tok_r, val_r)


_INV2PI = 0.15915493667125702
_TWOPI = 6.2831854820251465
_S1, _S2, _S3, _S4, _S5 = (
    0.9999846160704663,
    -0.16663261875795207,
    0.008312396647128057,
    -0.0001931637862847349,
    2.1733051646932733e-06,
)
_C0, _C1, _C2, _C3, _C4 = (
    0.9999710932183866,
    -0.49983759608552286,
    0.04152230455014086,
    -0.0013441068677407103,
    1.906521608691092e-05,
)


def _fast_sincos(ang):
    n = jnp.floor(ang * _INV2PI + 0.5)
    r = ang - n * _TWOPI
    r2 = r * r
    s = r * (_S1 + r2 * (_S2 + r2 * (_S3 + r2 * (_S4 + r2 * _S5))))
    c = _C0 + r2 * (_C1 + r2 * (_C2 + r2 * (_C3 + r2 * _C4)))
    return s, c


_NOH = 4 * (NFD + 1)
_KF = 256
_KEXT = 384


def _tc_dense_body(
    x_ref, ea_ref, eb_ref, wp_ref, w_ref, o_ref, feats_ref, tab_ref, wx_ref, ang_ref
):
    x = x_ref[:, :]

    @pl.when(pl.program_id(0) == 0)
    def _init():
        tab_ref[:, :] = jnp.zeros((_NOH, 4 * EMB_DIM), jnp.float32)
        tab_ref[0:5, 0:64] = ea_ref[:, :]
        tab_ref[5:10, 64:128] = eb_ref[:, :]
        tab_ref[10:15, 128:192] = ea_ref[:, :]
        tab_ref[15:20, 192:256] = eb_ref[:, :]
        tp = jnp.dot(tab_ref[:, :], w_ref[:, :], preferred_element_type=jnp.float32)
        wx_ref[0:_KF, :] = wp_ref[:, :].astype(jnp.bfloat16)
        wx_ref[_KF : _KF + _NOH, :] = tp.astype(jnp.bfloat16)
        wx_ref[_KF + _NOH :, :] = jnp.zeros(
            (_KEXT - _KF - _NOH, wx_ref.shape[1]), jnp.bfloat16
        )
        feats_ref[:, _KF + _NOH :] = jnp.zeros(
            (feats_ref.shape[0], _KEXT - _KF - _NOH), jnp.bfloat16
        )

    ncol = 4 * HALF + _NOH
    colf = lax.broadcasted_iota(jnp.int32, (8, ncol), 1)
    rowi = lax.broadcasted_iota(jnp.int32, (8, ncol), 0)
    jbr = colf & 3
    kf = (colf >> 2).astype(jnp.float32)
    fr = jnp.exp(kf * jnp.float32(-np.log(10000.0) / HALF))
    rowneed = jnp.where(
        jbr == 0, 0, jnp.where(jbr == 1, 4, jnp.where(jbr == 2, 1, 5))
    )
    ohc = colf - 4 * HALF
    ohb = ohc // (NFD + 1)
    ohneed = jnp.where(
        ohb == 0, 2, jnp.where(ohb == 1, 6, jnp.where(ohb == 2, 3, 7))
    )
    sel = jnp.where(colf < 4 * HALF, rowneed, ohneed)
    coef = jnp.where(colf < 4 * HALF, fr, jnp.float32(1.0))
    fmat = jnp.where(rowi == sel, coef, jnp.float32(0.0))
    ang_hi = jnp.dot(
        x,
        fmat[:, 0:32],
        preferred_element_type=jnp.float32,
        precision=lax.Precision.HIGHEST,
    )
    anglo_oh = jnp.dot(x, fmat[:, 32:], preferred_element_type=jnp.float32)
    ang_ref[:, 0:32] = ang_hi
    ang_ref[:, 32:] = anglo_oh[:, 0 : 4 * HALF - 32]
    s_all, c_all = _fast_sincos(ang_ref[:, :])
    feats_ref[:, 0 : 4 * HALF] = s_all.astype(jnp.bfloat16)
    feats_ref[:, 4 * HALF : 8 * HALF] = c_all.astype(jnp.bfloat16)
    ridx = ohc.astype(jnp.float32)[0:1, 4 * HALF :] - (
        (NFD + 1) * ohb.astype(jnp.float32)[0:1, 4 * HALF :]
    )
    feats_ref[:, _KF : _KF + _NOH] = jnp.where(
        anglo_oh[:, 4 * HALF - 32 :] == ridx, jnp.float32(1.0), jnp.float32(0.0)
    ).astype(jnp.bfloat16)

    o_ref[:, :] = jnp.dot(
        feats_ref[:, :], wx_ref[:, :], preferred_element_type=jnp.float32
    )


def _tc_dense(packed, emb_a, emb_b, w_p, w_t, block_rows=1024):
    n = packed.shape[0]
    proj = w_t.shape[1]
    grid = n // block_rows
    return pl.pallas_call(
        _tc_dense_body,
        grid=(grid,),
        in_specs=[
            pl.BlockSpec((block_rows, 8), lambda i: (i, 0)),
            pl.BlockSpec((NFD + 1, EMB_DIM), lambda i: (0, 0)),
            pl.BlockSpec((NFD + 1, EMB_DIM), lambda i: (0, 0)),
            pl.BlockSpec((4 * EMB_DIM, proj), lambda i: (0, 0)),
            pl.BlockSpec((4 * EMB_DIM, proj), lambda i: (0, 0)),
        ],
        out_specs=pl.BlockSpec((block_rows, proj), lambda i: (i, 0)),
        out_shape=jax.ShapeDtypeStruct((n, proj), jnp.float32),
        scratch_shapes=[
            pltpu.VMEM((block_rows, _KEXT), jnp.bfloat16),
            pltpu.VMEM((_NOH, 4 * EMB_DIM), jnp.float32),
            pltpu.VMEM((_KEXT, proj), jnp.bfloat16),
            pltpu.VMEM((block_rows, 4 * HALF), jnp.float32),
        ],
        compiler_params=pltpu.CompilerParams(
            dimension_semantics=("parallel",),
        ),
    )(packed, emb_a, emb_b, w_p, w_t)


def kernel(tokens, values, emb_a, emb_b, proj_w):
    b, s = tokens.shape[:2]
    groups = b // _LANES
    tok_r = tokens.reshape(groups, _LANES, s, 2).transpose(0, 3, 2, 1)
    val_r = values.reshape(groups, _LANES, s, 2).transpose(0, 3, 2, 1)
    sc_out = _sc_scan(tok_r, val_r)
    packed = sc_out.transpose(0, 4, 2, 1, 3).reshape(b * s, 8)
    w_t = proj_w.T
    w_p = (
        w_t.reshape(4, 2, HALF, -1)
        .transpose(1, 2, 0, 3)
        .reshape(4 * EMB_DIM, -1)
    )
    out = _tc_dense(packed, emb_a, emb_b, w_p, w_t)
    return out.reshape(b, s, proj_w.shape[0])


# speedup vs baseline: 1.1054x; 1.1054x over previous
import functools

import jax
import jax.numpy as jnp
import numpy as np
from jax import lax
from jax.experimental import pallas as pl
from jax.experimental.pallas import tpu as pltpu
from jax.experimental.pallas import tpu_sc as plsc

NFD = 4
MASK_ID = 1
SOS_ID = 2
EOS_ID = 3
EMB_DIM = 64
HALF = EMB_DIM // 2

_LANES = 16


def _sc_scan_body(tok_hbm, val_hbm, out_hbm, tok_v, val_v, out_v):
    info = plsc.get_sparse_core_info()
    nc = info.num_cores
    wid = lax.axis_index("s") * nc + lax.axis_index("c")
    num_workers = nc * info.num_subcores
    b_total = tok_hbm.shape[0]
    seq = tok_hbm.shape[1] // 2
    groups = b_total // _LANES
    g_per_w = groups // num_workers

    zf = jnp.zeros((_LANES,), jnp.float32)
    zi = jnp.zeros((_LANES,), jnp.int32)
    one_i = jnp.full((_LANES,), 1, jnp.int32)
    mask_i = jnp.full((_LANES,), MASK_ID, jnp.int32)
    nfd_i = jnp.full((_LANES,), NFD, jnp.int32)
    sos_i = jnp.full((_LANES,), SOS_ID, jnp.int32)
    eos_i = jnp.full((_LANES,), EOS_ID, jnp.int32)
    lane_iota = lax.iota(jnp.int32, _LANES)
    row_base = lane_iota * seq
    col_consts = [jnp.full((_LANES,), c, jnp.int32) for c in range(8)]

    for gi in range(g_per_w):
        g = wid * g_per_w + gi
        pltpu.sync_copy(tok_hbm.at[pl.ds(g * _LANES, _LANES)], tok_v)
        pltpu.sync_copy(val_hbm.at[pl.ds(g * _LANES, _LANES)], val_v)
        for i in range(2):

            def step(s, carry, i=i):
                unk, ck, ci = carry
                col_in = zi + (s * 2 + i)
                t = plsc.load_gather(tok_v, [lane_iota, col_in])
                vraw = plsc.load_gather(val_v, [lane_iota, col_in])
                special = t <= nfd_i
                tcv = jnp.where(special, t, nfd_i)
                v = jnp.where(special, zf, vraw)
                sm = special & (t != sos_i) & (t != eos_i)
                unk2 = unk | jnp.where(sm, one_i, zi)
                unkb = unk2 > zi
                ck2 = ck + v
                pk = jnp.where(unkb, zf, ck2)
                ci2 = jnp.where(sm, zf, ci + v)
                tk = jnp.where(unkb & (tcv == nfd_i), mask_i, tcv)
                row_out = row_base + s
                plsc.store_scatter(out_v, [row_out, col_consts[i * 4]], pk)
                plsc.store_scatter(out_v, [row_out, col_consts[i * 4 + 1]], ci2)
                plsc.store_scatter(
                    out_v, [row_out, col_consts[i * 4 + 2]], tk.astype(jnp.float32)
                )
                plsc.store_scatter(
                    out_v, [row_out, col_consts[i * 4 + 3]], tcv.astype(jnp.float32)
                )
                return (unk2, ck2, ci2)

            lax.fori_loop(0, seq, step, (zi, zf, zf))
        rows = _LANES * seq
        pltpu.sync_copy(out_v, out_hbm.at[pl.ds(g * rows, rows)])


def _sc_scan(tokens, values):
    b, seq, _ = tokens.shape
    mesh = plsc.VectorSubcoreMesh(core_axis_name="c", subcore_axis_name="s")
    fn = functools.partial(
        pl.kernel,
        mesh=mesh,
        out_type=jax.ShapeDtypeStruct((b * seq, 8), jnp.float32),
        scratch_types=[
            pltpu.VMEM((_LANES, seq * 2), jnp.int32),
            pltpu.VMEM((_LANES, seq * 2), jnp.float32),
            pltpu.VMEM((_LANES * seq, 8), jnp.float32),
        ],
        compiler_params=pltpu.CompilerParams(
            use_tc_tiling_on_sc=False, needs_layout_passes=False
        ),
    )(_sc_scan_body)
    return fn(tokens.reshape(b, seq * 2), values.reshape(b, seq * 2))


_INV2PI = 0.15915493667125702
_TWOPI = 6.2831854820251465
_S1, _S2, _S3, _S4, _S5 = (
    0.9999846160704663,
    -0.16663261875795207,
    0.008312396647128057,
    -0.0001931637862847349,
    2.1733051646932733e-06,
)
_C0, _C1, _C2, _C3, _C4 = (
    0.9999710932183866,
    -0.49983759608552286,
    0.04152230455014086,
    -0.0013441068677407103,
    1.906521608691092e-05,
)


def _fast_sincos(ang):
    n = jnp.floor(ang * _INV2PI + 0.5)
    r = ang - n * _TWOPI
    r2 = r * r
    s = r * (_S1 + r2 * (_S2 + r2 * (_S3 + r2 * (_S4 + r2 * _S5))))
    c = _C0 + r2 * (_C1 + r2 * (_C2 + r2 * (_C3 + r2 * _C4)))
    return s, c


_NOH = 4 * (NFD + 1)
_KF = 256
_KEXT = _KF + _NOH


def _tc_prep_body(ea_ref, eb_ref, wp_ref, w_ref, wx_ref, tab_ref):
    tab_ref[:, :] = jnp.zeros((_NOH, 4 * EMB_DIM), jnp.float32)
    tab_ref[0:5, 0:64] = ea_ref[:, :]
    tab_ref[5:10, 64:128] = eb_ref[:, :]
    tab_ref[10:15, 128:192] = ea_ref[:, :]
    tab_ref[15:20, 192:256] = eb_ref[:, :]
    tp = jnp.dot(tab_ref[:, :], w_ref[:, :], preferred_element_type=jnp.float32)
    wx_ref[0:_KF, :] = wp_ref[:, :].astype(jnp.bfloat16)
    wx_ref[_KF:_KEXT, :] = tp.astype(jnp.bfloat16)


def _tc_prep(emb_a, emb_b, w_p, w_t):
    proj = w_t.shape[1]
    return pl.pallas_call(
        _tc_prep_body,
        out_shape=jax.ShapeDtypeStruct((_KEXT, proj), jnp.bfloat16),
        scratch_shapes=[pltpu.VMEM((_NOH, 4 * EMB_DIM), jnp.float32)],
    )(emb_a, emb_b, w_p, w_t)


def _tc_dense_body(x_ref, wx_ref, o_ref, feats_ref, ang_ref):
    x = x_ref[:, :]

    ncol = 4 * HALF + _NOH
    colf = lax.broadcasted_iota(jnp.int32, (8, ncol), 1)
    rowi = lax.broadcasted_iota(jnp.int32, (8, ncol), 0)
    jbr = colf & 3
    kf = (colf >> 2).astype(jnp.float32)
    fr = jnp.exp(kf * jnp.float32(-np.log(10000.0) / HALF))
    rowneed = jnp.where(
        jbr == 0, 0, jnp.where(jbr == 1, 4, jnp.where(jbr == 2, 1, 5))
    )
    ohc = colf - 4 * HALF
    ohb = ohc // (NFD + 1)
    ohneed = jnp.where(
        ohb == 0, 2, jnp.where(ohb == 1, 6, jnp.where(ohb == 2, 3, 7))
    )
    sel = jnp.where(colf < 4 * HALF, rowneed, ohneed)
    coef = jnp.where(colf < 4 * HALF, fr, jnp.float32(1.0))
    fmat = jnp.where(rowi == sel, coef, jnp.float32(0.0))
    ang_hi = jnp.dot(
        x,
        fmat[:, 0:32],
        preferred_element_type=jnp.float32,
        precision=lax.Precision.HIGHEST,
    )
    anglo_oh = jnp.dot(x, fmat[:, 32:], preferred_element_type=jnp.float32)
    ang_ref[:, 0:32] = ang_hi
    ang_ref[:, 32:] = anglo_oh[:, 0 : 4 * HALF - 32]
    s_all, c_all = _fast_sincos(ang_ref[:, :])
    feats_ref[:, 0 : 4 * HALF] = s_all.astype(jnp.bfloat16)
    feats_ref[:, 4 * HALF : 8 * HALF] = c_all.astype(jnp.bfloat16)
    ridx = ohc.astype(jnp.float32)[0:1, 4 * HALF :] - (
        (NFD + 1) * ohb.astype(jnp.float32)[0:1, 4 * HALF :]
    )
    feats_ref[:, _KF:_KEXT] = jnp.where(
        anglo_oh[:, 4 * HALF - 32 :] == ridx, jnp.float32(1.0), jnp.float32(0.0)
    ).astype(jnp.bfloat16)

    o_ref[:, :] = jnp.dot(
        feats_ref[:, :], wx_ref[:, :], preferred_element_type=jnp.float32
    )


def _tc_dense(packed, wx, block_rows=1024):
    n = packed.shape[0]
    proj = wx.shape[1]
    grid = n // block_rows
    return pl.pallas_call(
        _tc_dense_body,
        grid=(grid,),
        in_specs=[
            pl.BlockSpec((block_rows, 8), lambda i: (i, 0)),
            pl.BlockSpec((_KEXT, proj), lambda i: (0, 0)),
        ],
        out_specs=pl.BlockSpec((block_rows, proj), lambda i: (i, 0)),
        out_shape=jax.ShapeDtypeStruct((n, proj), jnp.float32),
        scratch_shapes=[
            pltpu.VMEM((block_rows, _KEXT), jnp.bfloat16),
            pltpu.VMEM((block_rows, 4 * HALF), jnp.float32),
        ],
        compiler_params=pltpu.CompilerParams(
            dimension_semantics=("parallel",),
        ),
    )(packed, wx)


def kernel(tokens, values, emb_a, emb_b, proj_w):
    b, s = tokens.shape[:2]
    packed = _sc_scan(tokens, values)
    w_t = proj_w.T
    w_p = (
        w_t.reshape(4, 2, HALF, -1)
        .transpose(1, 2, 0, 3)
        .reshape(4 * EMB_DIM, -1)
    )
    wx = _tc_prep(emb_a, emb_b, w_p, w_t)
    out = _tc_dense(packed, wx)
    return out.reshape(b, s, proj_w.shape[0])


# speedup vs baseline: 1.1797x; 1.0673x over previous
import functools

import jax
import jax.numpy as jnp
import numpy as np
from jax import lax
from jax.experimental import pallas as pl
from jax.experimental.pallas import tpu as pltpu
from jax.experimental.pallas import tpu_sc as plsc

NFD = 4
MASK_ID = 1
SOS_ID = 2
EOS_ID = 3
EMB_DIM = 64
HALF = EMB_DIM // 2

_LANES = 16


def _sc_scan_body(tok_hbm, val_hbm, out_hbm, tok_v, val_v, out_v):
    info = plsc.get_sparse_core_info()
    nc = info.num_cores
    wid = lax.axis_index("s") * nc + lax.axis_index("c")
    num_workers = nc * info.num_subcores
    b_total = tok_hbm.shape[0]
    seq = tok_hbm.shape[1] // 2
    groups = b_total // _LANES
    g_per_w = groups // num_workers

    zf = jnp.zeros((_LANES,), jnp.float32)
    zi = jnp.zeros((_LANES,), jnp.int32)
    one_i = jnp.full((_LANES,), 1, jnp.int32)
    mask_i = jnp.full((_LANES,), MASK_ID, jnp.int32)
    nfd_i = jnp.full((_LANES,), NFD, jnp.int32)
    sos_i = jnp.full((_LANES,), SOS_ID, jnp.int32)
    eos_i = jnp.full((_LANES,), EOS_ID, jnp.int32)
    lane_iota = lax.iota(jnp.int32, _LANES)
    row_base = lane_iota * seq
    col_consts = [jnp.full((_LANES,), c, jnp.int32) for c in range(8)]

    for gi in range(g_per_w):
        g = wid * g_per_w + gi
        pltpu.sync_copy(tok_hbm.at[pl.ds(g * _LANES, _LANES)], tok_v)
        pltpu.sync_copy(val_hbm.at[pl.ds(g * _LANES, _LANES)], val_v)
        for i in range(2):

            def step(s, carry, i=i):
                unk, ck, ci = carry
                col_in = zi + (s * 2 + i)
                t = plsc.load_gather(tok_v, [lane_iota, col_in])
                vraw = plsc.load_gather(val_v, [lane_iota, col_in])
                special = t <= nfd_i
                tcv = jnp.where(special, t, nfd_i)
                v = jnp.where(special, zf, vraw)
                sm = special & (t != sos_i) & (t != eos_i)
                unk2 = unk | jnp.where(sm, one_i, zi)
                unkb = unk2 > zi
                ck2 = ck + v
                pk = jnp.where(unkb, zf, ck2)
                ci2 = jnp.where(sm, zf, ci + v)
                tk = jnp.where(unkb & (tcv == nfd_i), mask_i, tcv)
                row_out = row_base + s
                plsc.store_scatter(out_v, [row_out, col_consts[i * 4]], pk)
                plsc.store_scatter(out_v, [row_out, col_consts[i * 4 + 1]], ci2)
                plsc.store_scatter(
                    out_v, [row_out, col_consts[i * 4 + 2]], tk.astype(jnp.float32)
                )
                plsc.store_scatter(
                    out_v, [row_out, col_consts[i * 4 + 3]], tcv.astype(jnp.float32)
                )
                return (unk2, ck2, ci2)

            lax.fori_loop(0, seq, step, (zi, zf, zf), unroll=4)
        rows = _LANES * seq
        pltpu.sync_copy(out_v, out_hbm.at[pl.ds(g * rows, rows)])


def _sc_scan(tokens, values):
    b, seq, _ = tokens.shape
    mesh = plsc.VectorSubcoreMesh(core_axis_name="c", subcore_axis_name="s")
    fn = functools.partial(
        pl.kernel,
        mesh=mesh,
        out_type=jax.ShapeDtypeStruct((b * seq, 8), jnp.float32),
        scratch_types=[
            pltpu.VMEM((_LANES, seq * 2), jnp.int32),
            pltpu.VMEM((_LANES, seq * 2), jnp.float32),
            pltpu.VMEM((_LANES * seq, 8), jnp.float32),
        ],
        compiler_params=pltpu.CompilerParams(
            use_tc_tiling_on_sc=False, needs_layout_passes=False
        ),
    )(_sc_scan_body)
    return fn(tokens.reshape(b, seq * 2), values.reshape(b, seq * 2))


_INV2PI = 0.15915493667125702
_TWOPI = 6.2831854820251465
_S1, _S2, _S3, _S4, _S5 = (
    0.9999846160704663,
    -0.16663261875795207,
    0.008312396647128057,
    -0.0001931637862847349,
    2.1733051646932733e-06,
)
_C0, _C1, _C2, _C3, _C4 = (
    0.9999710932183866,
    -0.49983759608552286,
    0.04152230455014086,
    -0.0013441068677407103,
    1.906521608691092e-05,
)


def _fast_sincos(ang):
    n = jnp.floor(ang * _INV2PI + 0.5)
    r = ang - n * _TWOPI
    r2 = r * r
    s = r * (_S1 + r2 * (_S2 + r2 * (_S3 + r2 * (_S4 + r2 * _S5))))
    c = _C0 + r2 * (_C1 + r2 * (_C2 + r2 * (_C3 + r2 * _C4)))
    return s, c


_NOH = 4 * (NFD + 1)
_KF = 256
_KEXT = _KF + _NOH


def _tc_prep_body(ea_ref, eb_ref, wp_ref, w_ref, wx_ref, tab_ref):
    tab_ref[:, :] = jnp.zeros((_NOH, 4 * EMB_DIM), jnp.float32)
    tab_ref[0:5, 0:64] = ea_ref[:, :]
    tab_ref[5:10, 64:128] = eb_ref[:, :]
    tab_ref[10:15, 128:192] = ea_ref[:, :]
    tab_ref[15:20, 192:256] = eb_ref[:, :]
    tp = jnp.dot(tab_ref[:, :], w_ref[:, :], preferred_element_type=jnp.float32)
    wx_ref[0:_KF, :] = wp_ref[:, :].astype(jnp.bfloat16)
    wx_ref[_KF:_KEXT, :] = tp.astype(jnp.bfloat16)


def _tc_prep(emb_a, emb_b, w_p, w_t):
    proj = w_t.shape[1]
    return pl.pallas_call(
        _tc_prep_body,
        out_shape=jax.ShapeDtypeStruct((_KEXT, proj), jnp.bfloat16),
        scratch_shapes=[pltpu.VMEM((_NOH, 4 * EMB_DIM), jnp.float32)],
    )(emb_a, emb_b, w_p, w_t)


def _tc_dense_body(x_ref, wx_ref, o_ref, feats_ref, ang_ref):
    x = x_ref[:, :]

    ncol = 4 * HALF + _NOH
    colf = lax.broadcasted_iota(jnp.int32, (8, ncol), 1)
    rowi = lax.broadcasted_iota(jnp.int32, (8, ncol), 0)
    jbr = colf & 3
    kf = (colf >> 2).astype(jnp.float32)
    fr = jnp.exp(kf * jnp.float32(-np.log(10000.0) / HALF))
    rowneed = jnp.where(
        jbr == 0, 0, jnp.where(jbr == 1, 4, jnp.where(jbr == 2, 1, 5))
    )
    ohc = colf - 4 * HALF
    ohb = ohc // (NFD + 1)
    ohneed = jnp.where(
        ohb == 0, 2, jnp.where(ohb == 1, 6, jnp.where(ohb == 2, 3, 7))
    )
    sel = jnp.where(colf < 4 * HALF, rowneed, ohneed)
    coef = jnp.where(colf < 4 * HALF, fr, jnp.float32(1.0))
    fmat = jnp.where(rowi == sel, coef, jnp.float32(0.0))
    ang_hi = jnp.dot(
        x,
        fmat[:, 0:32],
        preferred_element_type=jnp.float32,
        precision=lax.Precision.HIGHEST,
    )
    anglo_oh = jnp.dot(x, fmat[:, 32:], preferred_element_type=jnp.float32)
    ang_ref[:, 0:32] = ang_hi
    ang_ref[:, 32:] = anglo_oh[:, 0 : 4 * HALF - 32]
    s_all, c_all = _fast_sincos(ang_ref[:, :])
    feats_ref[:, 0 : 4 * HALF] = s_all.astype(jnp.bfloat16)
    feats_ref[:, 4 * HALF : 8 * HALF] = c_all.astype(jnp.bfloat16)
    ridx = ohc.astype(jnp.float32)[0:1, 4 * HALF :] - (
        (NFD + 1) * ohb.astype(jnp.float32)[0:1, 4 * HALF :]
    )
    feats_ref[:, _KF:_KEXT] = jnp.where(
        anglo_oh[:, 4 * HALF - 32 :] == ridx, jnp.float32(1.0), jnp.float32(0.0)
    ).astype(jnp.bfloat16)

    o_ref[:, :] = jnp.dot(
        feats_ref[:, :], wx_ref[:, :], preferred_element_type=jnp.float32
    )


def _tc_dense(packed, wx, block_rows=2048):
    n = packed.shape[0]
    proj = wx.shape[1]
    grid = n // block_rows
    return pl.pallas_call(
        _tc_dense_body,
        grid=(grid,),
        in_specs=[
            pl.BlockSpec((block_rows, 8), lambda i: (i, 0)),
            pl.BlockSpec((_KEXT, proj), lambda i: (0, 0)),
        ],
        out_specs=pl.BlockSpec((block_rows, proj), lambda i: (i, 0)),
        out_shape=jax.ShapeDtypeStruct((n, proj), jnp.float32),
        scratch_shapes=[
            pltpu.VMEM((block_rows, _KEXT), jnp.bfloat16),
            pltpu.VMEM((block_rows, 4 * HALF), jnp.float32),
        ],
        compiler_params=pltpu.CompilerParams(
            dimension_semantics=("parallel",),
        ),
    )(packed, wx)


def kernel(tokens, values, emb_a, emb_b, proj_w):
    b, s = tokens.shape[:2]
    packed = _sc_scan(tokens, values)
    w_t = proj_w.T
    w_p = (
        w_t.reshape(4, 2, HALF, -1)
        .transpose(1, 2, 0, 3)
        .reshape(4 * EMB_DIM, -1)
    )
    wx = _tc_prep(emb_a, emb_b, w_p, w_t)
    out = _tc_dense(packed, wx)
    return out.reshape(b, s, proj_w.shape[0])
